# Initial kernel scaffold; baseline (speedup 1.0000x reference)
import jax
import jax.numpy as jnp
from jax.experimental import pallas as pl


def kernel(x, edge_index, edge_attr, params):
    raise NotImplementedError("write your pallas kernel here")


# speedup vs baseline: 2.9264x; 2.9264x over previous
import functools

import jax
import jax.numpy as jnp
from jax import lax
from jax.experimental import pallas as pl
from jax.experimental.pallas import tpu as pltpu
from jax.experimental.pallas import tpu_sc as plsc

NC = 2
NS = 16
NW = NC * NS
CHUNK = 128
GK = 4


def _mesh():
    return plsc.VectorSubcoreMesh(core_axis_name="c", subcore_axis_name="s")


_SC_PARAMS = pltpu.CompilerParams(use_tc_tiling_on_sc=False)


def _sc_gather2(npad, d, epad):
    per_w = epad // NW
    n_chunks = per_w // CHUNK
    n_groups = n_chunks // GK
    group = GK * CHUNK

    @functools.partial(
        pl.kernel,
        out_type=(jax.ShapeDtypeStruct((epad, d), jnp.float32),
                  jax.ShapeDtypeStruct((epad, d), jnp.float32)),
        mesh=_mesh(), compiler_params=_SC_PARAMS,
        scratch_types=[
            pltpu.VMEM((n_chunks, CHUNK), jnp.int32),
            pltpu.VMEM((n_chunks, CHUNK), jnp.int32),
            pltpu.VMEM((group, d), jnp.float32),
            pltpu.VMEM((group, d), jnp.float32),
            pltpu.SemaphoreType.DMA,
        ],
    )
    def k(table, src2, dst2, hs, hd, src_v, dst_v, rs_v, rd_v, sem):
        c = lax.axis_index("c")
        s = lax.axis_index("s")
        w = s * NC + c
        pltpu.sync_copy(src2.at[pl.ds(w * n_chunks, n_chunks)], src_v)
        pltpu.sync_copy(dst2.at[pl.ds(w * n_chunks, n_chunks)], dst_v)
        base = w * per_w

        def body(g, carry):
            j0 = g * GK
            descs = []
            for u in range(GK):
                descs.append(pltpu.async_copy(
                    table.at[src_v.at[j0 + u]],
                    rs_v.at[pl.ds(u * CHUNK, CHUNK)], sem))
                descs.append(pltpu.async_copy(
                    table.at[dst_v.at[j0 + u]],
                    rd_v.at[pl.ds(u * CHUNK, CHUNK)], sem))
            for dsc in descs:
                dsc.wait()
            pltpu.sync_copy(rs_v, hs.at[pl.ds(base + g * group, group)])
            pltpu.sync_copy(rd_v, hd.at[pl.ds(base + g * group, group)])
            return carry

        lax.fori_loop(0, n_groups, body, None)

    return k


def _sc_scatter_add(npad, width, epad, gather_table=False):
    per_w = epad // NW
    n_chunks = per_w // CHUNK
    rpt = npad // NS
    rr = 40
    nb = n_chunks // rr

    out_t = jax.ShapeDtypeStruct((NC * npad, width), jnp.float32)
    scratch = [
        pltpu.VMEM((rr, CHUNK), jnp.int32),
        pltpu.VMEM((rr, CHUNK), jnp.int32),
        pltpu.VMEM((2, CHUNK, width), jnp.float32),
        pltpu.VMEM_SHARED((npad, width), jnp.float32),
        pltpu.SemaphoreType.DMA,
        pltpu.SemaphoreType.DMA,
    ]

    @functools.partial(pl.kernel, out_type=out_t, mesh=_mesh(),
                       compiler_params=_SC_PARAMS, scratch_types=scratch)
    def k(src_data, dst2, zeros, src2, out, dst_v, src_v, buf, acc,
          sem0, sem1):
        sems = (sem0, sem1)
        c = lax.axis_index("c")
        s = lax.axis_index("s")
        w = s * NC + c
        pltpu.sync_copy(zeros.at[pl.ds(s * rpt, rpt)],
                        acc.at[pl.ds(s * rpt, rpt)])
        plsc.subcore_barrier()
        base = w * per_w

        def start(jl, jg, b):
            if gather_table:
                pltpu.async_copy(src_data.at[src_v.at[jl]], buf.at[b],
                                 sems[b])
            else:
                pltpu.async_copy(
                    src_data.at[pl.ds(base + jg * CHUNK, CHUNK)],
                    buf.at[b], sems[b])

        def outer(ob, carry):
            pltpu.sync_copy(
                dst2.at[pl.ds(w * n_chunks + ob * rr, rr)], dst_v)
            if gather_table:
                pltpu.sync_copy(
                    src2.at[pl.ds(w * n_chunks + ob * rr, rr)], src_v)
            start(0, ob * rr, 0)
            start(1, ob * rr + 1, 1)

            def body(g, carry2):
                for b in range(2):
                    jl = g * 2 + b
                    pltpu.make_async_copy(src_data.at[pl.ds(0, CHUNK)],
                                          buf.at[b], sems[b]).wait()
                    pltpu.sync_copy(buf.at[b], acc.at[dst_v.at[jl]],
                                    add=True)

                    @pl.when(jl + 2 < rr)
                    def _():
                        start(jl + 2, ob * rr + jl + 2, b)
                return carry2

            lax.fori_loop(0, rr // 2, body, None)
            return carry

        lax.fori_loop(0, nb, outer, None)
        plsc.subcore_barrier()
        pltpu.sync_copy(acc.at[pl.ds(s * rpt, rpt)],
                        out.at[pl.ds(c * npad + s * rpt, rpt)])

    return k


def _sc_count(npad, epad):
    per_w = epad // NW
    n_chunks = per_w // CHUNK
    rpt = npad // NS
    width = 16

    @functools.partial(
        pl.kernel,
        out_type=jax.ShapeDtypeStruct((NC * npad, width), jnp.float32),
        mesh=_mesh(), compiler_params=_SC_PARAMS,
        scratch_types=[
            pltpu.VMEM((n_chunks, CHUNK), jnp.int32),
            pltpu.VMEM((CHUNK, width), jnp.float32),
            pltpu.VMEM_SHARED((npad, width), jnp.float32),
        ],
    )
    def k(dst2, zeros, ones, out, dst_v, ones_v, acc):
        c = lax.axis_index("c")
        s = lax.axis_index("s")
        w = s * NC + c
        pltpu.sync_copy(dst2.at[pl.ds(w * n_chunks, n_chunks)], dst_v)
        pltpu.sync_copy(ones, ones_v)
        pltpu.sync_copy(zeros.at[pl.ds(s * rpt, rpt)],
                        acc.at[pl.ds(s * rpt, rpt)])
        plsc.subcore_barrier()

        def body(j, carry):
            pltpu.sync_copy(ones_v, acc.at[dst_v.at[j]], add=True)
            return carry

        lax.fori_loop(0, n_chunks, body, None)
        plsc.subcore_barrier()
        pltpu.sync_copy(acc.at[pl.ds(s * rpt, rpt)],
                        out.at[pl.ds(c * npad + s * rpt, rpt)])

    return k


def _tc_edge_mlp(epad, dh, de, be):
    grid = epad // be

    def body(hs, hd, ef, a, b, cc, b1, w2, b2, out):
        z = jnp.dot(hs[...], a[...], preferred_element_type=jnp.float32)
        z += jnp.dot(hd[...], b[...], preferred_element_type=jnp.float32)
        z += jnp.dot(ef[...], cc[...], preferred_element_type=jnp.float32)
        z = jnp.maximum(z + b1[...], 0.0)
        out[...] = jnp.dot(z, w2[...],
                           preferred_element_type=jnp.float32) + b2[...]

    def full(shape):
        return pl.BlockSpec(shape, lambda i: (0, 0))

    def make(a, b, cc, b1, w2, b2):
        call = pl.pallas_call(
            body,
            grid=(grid,),
            in_specs=[
                pl.BlockSpec((be, dh), lambda i: (i, 0)),
                pl.BlockSpec((be, dh), lambda i: (i, 0)),
                pl.BlockSpec((be, de), lambda i: (i, 0)),
                full(a.shape), full(b.shape), full(cc.shape),
                full(b1.shape), full(w2.shape), full(b2.shape),
            ],
            out_specs=pl.BlockSpec((be, 32), lambda i: (i, 0)),
            out_shape=jax.ShapeDtypeStruct((epad, 32), jnp.float32),
        )
        return lambda hs, hd, ef: call(hs, hd, ef, a, b, cc, b1, w2, b2)

    return make


def _tc_node_mlp(npad, dh, bn):
    grid = npad // bn

    def body(h, aggp, d1, d2, b1, w2, b2, out):
        agg = aggp[0] + aggp[1]
        z = jnp.dot(h[...], d1[...], preferred_element_type=jnp.float32)
        z += jnp.dot(agg, d2[...], preferred_element_type=jnp.float32)
        z = jnp.maximum(z + b1[...], 0.0)
        out[...] = jnp.dot(z, w2[...],
                           preferred_element_type=jnp.float32) + b2[...]

    def full(shape):
        return pl.BlockSpec(shape, lambda i: tuple(0 for _ in shape))

    def make(d1, d2, b1, w2, b2):
        call = pl.pallas_call(
            body,
            grid=(grid,),
            in_specs=[
                pl.BlockSpec((bn, dh), lambda i: (i, 0)),
                pl.BlockSpec((NC, bn, 32), lambda i: (0, i, 0)),
                full(d1.shape), full(d2.shape), full(b1.shape),
                full(w2.shape), full(b2.shape),
            ],
            out_specs=pl.BlockSpec((bn, 32), lambda i: (i, 0)),
            out_shape=jax.ShapeDtypeStruct((npad, 32), jnp.float32),
        )
        return lambda h, aggp: call(h, aggp, d1, d2, b1, w2, b2)

    return make


def _tc_assemble(npad, bn):
    grid = npad // bn

    def body(sp, cp, h0, out):
        cnt = jnp.maximum((cp[0] + cp[1])[:, 0:1], 1.0)
        hm = (sp[0] + sp[1]) / cnt
        out[...] = jnp.concatenate(
            [hm, h0[:, 0:2], h0[:, 3:6], jnp.zeros((bn, 11), jnp.float32)],
            axis=1)

    return pl.pallas_call(
        body,
        grid=(grid,),
        in_specs=[
            pl.BlockSpec((NC, bn, 32), lambda i: (0, i, 0)),
            pl.BlockSpec((NC, bn, 16), lambda i: (0, i, 0)),
            pl.BlockSpec((bn, 16), lambda i: (i, 0)),
        ],
        out_specs=pl.BlockSpec((bn, 48), lambda i: (i, 0)),
        out_shape=jax.ShapeDtypeStruct((npad, 48), jnp.float32),
    )


def _tc_final(npad, bn):
    grid = npad // bn

    def body(sp, cp, h0, wa, wb, bd, out):
        cnt = jnp.maximum((cp[0] + cp[1])[:, 0:1], 1.0)
        hm = (sp[0] + sp[1]) / cnt
        z = jnp.dot(hm, wa[...], preferred_element_type=jnp.float32)
        z += jnp.dot(h0[:, 0:2], wb[...], preferred_element_type=jnp.float32)
        out[...] = z + bd[...]

    def full(shape):
        return pl.BlockSpec(shape, lambda i: (0, 0))

    def make(wa, wb, bd):
        call = pl.pallas_call(
            body,
            grid=(grid,),
            in_specs=[
                pl.BlockSpec((NC, bn, 32), lambda i: (0, i, 0)),
                pl.BlockSpec((NC, bn, 16), lambda i: (0, i, 0)),
                pl.BlockSpec((bn, 16), lambda i: (i, 0)),
                full(wa.shape), full(wb.shape), full(bd.shape),
            ],
            out_specs=pl.BlockSpec((bn, 8), lambda i: (i, 0)),
            out_shape=jax.ShapeDtypeStruct((npad, 8), jnp.float32),
        )
        return lambda sp, cp, h0: call(sp, cp, h0, wa, wb, bd)

    return make


def _pad_rows(w, rows):
    return jnp.concatenate(
        [w, jnp.zeros((rows - w.shape[0], w.shape[1]), w.dtype)], axis=0)


def kernel(x, edge_index, edge_attr, params):
    n = x.shape[0]
    e = edge_index.shape[1]
    npad = ((n + 16) + 127) // 128 * 128
    epad = -(-e // (NW * CHUNK * 8)) * (NW * CHUNK * 8)
    n_chunks_tot = epad // CHUNK
    bn = npad // 8
    be = 4096

    src = edge_index[0].astype(jnp.int32)
    dst = edge_index[1].astype(jnp.int32)
    src2 = jnp.full((epad,), 0, jnp.int32).at[:e].set(src).reshape(
        n_chunks_tot, CHUNK)
    dst2 = jnp.full((epad,), n, jnp.int32).at[:e].set(dst).reshape(
        n_chunks_tot, CHUNK)
    ea = _pad_rows(edge_attr.astype(jnp.float32), epad)

    z32 = jnp.zeros((npad, 32), jnp.float32)
    z16 = jnp.zeros((npad, 16), jnp.float32)
    ones16 = jnp.ones((CHUNK, 16), jnp.float32)

    h0p = jnp.zeros((npad, 16), jnp.float32).at[:n, :6].set(
        x.astype(jnp.float32))

    p0, p1 = params["proc0"], params["proc1"]
    row = lambda v: v.reshape(1, -1).astype(jnp.float32)
    f32 = lambda v: v.astype(jnp.float32)

    a0 = _pad_rows(f32(p0["We1"][0:6]), 16)
    b0 = _pad_rows(f32(p0["We1"][6:12]), 16)
    c0 = f32(p0["We1"][12:16])
    d1_0 = _pad_rows(f32(p0["Wn1"][0:6]), 16)
    d2_0 = f32(p0["Wn1"][6:38])
    a1 = _pad_rows(f32(p1["We1"][0:37]), 48)
    b1w = _pad_rows(f32(p1["We1"][37:74]), 48)
    c1 = f32(p1["We1"][74:106])
    d1_1 = _pad_rows(f32(p1["Wn1"][0:37]), 48)
    d2_1 = f32(p1["Wn1"][37:69])
    wda = jnp.concatenate(
        [f32(params["Wd"][0:32]), jnp.zeros((32, 5), jnp.float32)], axis=1)
    wdb = jnp.concatenate(
        [f32(params["Wd"][32:34]), jnp.zeros((2, 5), jnp.float32)], axis=1)
    bdp = jnp.concatenate(
        [row(params["bd"]), jnp.zeros((1, 5), jnp.float32)], axis=1)

    gather16 = _sc_gather2(npad, 16, epad)
    gather48 = _sc_gather2(npad, 48, epad)
    scat_vals = _sc_scatter_add(npad, 32, epad, gather_table=False)
    scat_gath = _sc_scatter_add(npad, 32, epad, gather_table=True)
    count_k = _sc_count(npad, epad)
    edge0 = _tc_edge_mlp(epad, 16, 4, be)(
        a0, b0, c0, row(p0["be1"]), f32(p0["We2"]), row(p0["be2"]))
    edge1 = _tc_edge_mlp(epad, 48, 32, be)(
        a1, b1w, c1, row(p1["be1"]), f32(p1["We2"]), row(p1["be2"]))
    node0 = _tc_node_mlp(npad, 16, bn)(
        d1_0, d2_0, row(p0["bn1"]), f32(p0["Wn2"]), row(p0["bn2"]))
    node1 = _tc_node_mlp(npad, 48, bn)(
        d1_1, d2_1, row(p1["bn1"]), f32(p1["Wn2"]), row(p1["bn2"]))
    assemble = _tc_assemble(npad, bn)
    final = _tc_final(npad, bn)(wda, wdb, bdp)

    as3 = lambda v, w: v.reshape(NC, npad, w)

    hs0, hd0 = gather16(h0p, src2, dst2)
    e1 = edge0(hs0, hd0, ea)
    agg0 = scat_vals(e1, dst2, z32, src2)
    h1 = node0(h0p, as3(agg0, 32))
    cnt = count_k(dst2, z16, ones16)
    s0 = scat_gath(h1, dst2, z32, src2)
    h1cat = assemble(as3(s0, 32), as3(cnt, 16), h0p)
    hs1, hd1 = gather48(h1cat, src2, dst2)
    e2 = edge1(hs1, hd1, e1)
    agg1 = scat_vals(e2, dst2, z32, src2)
    h2 = node1(h1cat, as3(agg1, 32))
    s1 = scat_gath(h2, dst2, z32, src2)
    out = final(as3(s1, 32), as3(cnt, 16), h0p)
    return out[:n, :3]


# speedup vs baseline: 2.9901x; 1.0218x over previous
import functools

import jax
import jax.numpy as jnp
from jax import lax
from jax.experimental import pallas as pl
from jax.experimental.pallas import tpu as pltpu
from jax.experimental.pallas import tpu_sc as plsc

NC = 2
NS = 16
NW = NC * NS
CHUNK = 128
GK = 4


def _mesh():
    return plsc.VectorSubcoreMesh(core_axis_name="c", subcore_axis_name="s")


_SC_PARAMS = pltpu.CompilerParams(use_tc_tiling_on_sc=False)


def _sc_gather2(npad, d, epad):
    per_w = epad // NW
    n_chunks = per_w // CHUNK

    @functools.partial(
        pl.kernel,
        out_type=(jax.ShapeDtypeStruct((epad, d), jnp.float32),
                  jax.ShapeDtypeStruct((epad, d), jnp.float32)),
        mesh=_mesh(), compiler_params=_SC_PARAMS,
        scratch_types=[
            pltpu.VMEM((n_chunks, CHUNK), jnp.int32),
            pltpu.VMEM((n_chunks, CHUNK), jnp.int32),
            pltpu.VMEM((4, CHUNK, d), jnp.float32),
            pltpu.VMEM((4, CHUNK, d), jnp.float32),
            [pltpu.SemaphoreType.DMA] * 4,
            [pltpu.SemaphoreType.DMA] * 4,
        ],
    )
    def k(table, src2, dst2, hs, hd, src_v, dst_v, rs_v, rd_v, gsem, wsem):
        c = lax.axis_index("c")
        s = lax.axis_index("s")
        w = s * NC + c
        pltpu.sync_copy(src2.at[pl.ds(w * n_chunks, n_chunks)], src_v)
        pltpu.sync_copy(dst2.at[pl.ds(w * n_chunks, n_chunks)], dst_v)
        base = w * per_w

        def fire_gather(j, slot):
            pltpu.async_copy(table.at[src_v.at[j]], rs_v.at[slot],
                             gsem[slot])
            pltpu.async_copy(table.at[dst_v.at[j]], rd_v.at[slot],
                             gsem[slot])

        fire_gather(0, 0)
        fire_gather(1, 1)

        def body(og, carry):
            for b in range(4):
                j = og * 4 + b
                pltpu.make_async_copy(table.at[pl.ds(0, CHUNK)],
                                      rs_v.at[b], gsem[b]).wait()
                pltpu.make_async_copy(table.at[pl.ds(0, CHUNK)],
                                      rd_v.at[b], gsem[b]).wait()
                pltpu.async_copy(rs_v.at[b],
                                 hs.at[pl.ds(base + j * CHUNK, CHUNK)],
                                 wsem[b])
                pltpu.async_copy(rd_v.at[b],
                                 hd.at[pl.ds(base + j * CHUNK, CHUNK)],
                                 wsem[b])
                b2 = (b + 2) % 4

                @pl.when(j + 2 < n_chunks)
                def _():
                    @pl.when(j >= 2)
                    def _():
                        pltpu.make_async_copy(
                            hs.at[pl.ds(base, CHUNK)], rs_v.at[b2],
                            wsem[b2]).wait()
                        pltpu.make_async_copy(
                            hd.at[pl.ds(base, CHUNK)], rd_v.at[b2],
                            wsem[b2]).wait()
                    fire_gather(j + 2, b2)
            return carry

        lax.fori_loop(0, n_chunks // 4, body, None)
        for b in range(4):
            pltpu.make_async_copy(hs.at[pl.ds(base, CHUNK)], rs_v.at[b],
                                  wsem[b]).wait()
            pltpu.make_async_copy(hd.at[pl.ds(base, CHUNK)], rd_v.at[b],
                                  wsem[b]).wait()

    return k


def _sc_scatter_add(npad, width, epad, gather_table=False):
    per_w = epad // NW
    n_chunks = per_w // CHUNK
    rpt = npad // NS
    rr = 40
    nb = n_chunks // rr

    out_t = jax.ShapeDtypeStruct((NC * npad, width), jnp.float32)
    scratch = [
        pltpu.VMEM((rr, CHUNK), jnp.int32),
        pltpu.VMEM((rr, CHUNK), jnp.int32),
        pltpu.VMEM((4, CHUNK, width), jnp.float32),
        pltpu.VMEM_SHARED((npad, width), jnp.float32),
        [pltpu.SemaphoreType.DMA] * 4,
        [pltpu.SemaphoreType.DMA] * 4,
    ]

    @functools.partial(pl.kernel, out_type=out_t, mesh=_mesh(),
                       compiler_params=_SC_PARAMS, scratch_types=scratch)
    def k(src_data, dst2, zeros, src2, out, dst_v, src_v, buf, acc,
          lsem, ssem):
        c = lax.axis_index("c")
        s = lax.axis_index("s")
        w = s * NC + c
        pltpu.sync_copy(zeros.at[pl.ds(s * rpt, rpt)],
                        acc.at[pl.ds(s * rpt, rpt)])
        plsc.subcore_barrier()
        base = w * per_w

        def start(jl, jg, b):
            if gather_table:
                pltpu.async_copy(src_data.at[src_v.at[jl]], buf.at[b],
                                 lsem[b])
            else:
                pltpu.async_copy(
                    src_data.at[pl.ds(base + jg * CHUNK, CHUNK)],
                    buf.at[b], lsem[b])

        def outer(ob, carry):
            pltpu.sync_copy(
                dst2.at[pl.ds(w * n_chunks + ob * rr, rr)], dst_v)
            if gather_table:
                pltpu.sync_copy(
                    src2.at[pl.ds(w * n_chunks + ob * rr, rr)], src_v)
            start(0, ob * rr, 0)
            start(1, ob * rr + 1, 1)

            def body(g, carry2):
                for b in range(4):
                    jl = g * 4 + b
                    b2 = (b + 2) % 4
                    pltpu.make_async_copy(src_data.at[pl.ds(0, CHUNK)],
                                          buf.at[b], lsem[b]).wait()
                    pltpu.async_copy(buf.at[b], acc.at[dst_v.at[jl]],
                                     ssem[b], add=True)

                    @pl.when(jl + 2 < rr)
                    def _():
                        @pl.when(jl >= 2)
                        def _():
                            pltpu.make_async_copy(
                                src_data.at[pl.ds(0, CHUNK)],
                                buf.at[b2], ssem[b2]).wait()
                        start(jl + 2, ob * rr + jl + 2, b2)
                return carry2

            lax.fori_loop(0, rr // 4, body, None)
            for b in range(4):
                pltpu.make_async_copy(src_data.at[pl.ds(0, CHUNK)],
                                      buf.at[b], ssem[b]).wait()
            return carry

        lax.fori_loop(0, nb, outer, None)
        plsc.subcore_barrier()
        pltpu.sync_copy(acc.at[pl.ds(s * rpt, rpt)],
                        out.at[pl.ds(c * npad + s * rpt, rpt)])

    return k


def _sc_count(npad, epad):
    per_w = epad // NW
    n_chunks = per_w // CHUNK
    rpt = npad // NS
    width = 16

    @functools.partial(
        pl.kernel,
        out_type=jax.ShapeDtypeStruct((NC * npad, width), jnp.float32),
        mesh=_mesh(), compiler_params=_SC_PARAMS,
        scratch_types=[
            pltpu.VMEM((n_chunks, CHUNK), jnp.int32),
            pltpu.VMEM((CHUNK, width), jnp.float32),
            pltpu.VMEM_SHARED((npad, width), jnp.float32),
            pltpu.SemaphoreType.DMA,
        ],
    )
    def k(dst2, zeros, ones, out, dst_v, ones_v, acc, sem):
        c = lax.axis_index("c")
        s = lax.axis_index("s")
        w = s * NC + c
        pltpu.sync_copy(dst2.at[pl.ds(w * n_chunks, n_chunks)], dst_v)
        pltpu.sync_copy(ones, ones_v)
        pltpu.sync_copy(zeros.at[pl.ds(s * rpt, rpt)],
                        acc.at[pl.ds(s * rpt, rpt)])
        plsc.subcore_barrier()

        def body(g, carry):
            for u in range(8):
                pltpu.async_copy(ones_v, acc.at[dst_v.at[g * 8 + u]], sem,
                                 add=True)
            for _ in range(8):
                pltpu.make_async_copy(ones, ones_v, sem).wait()
            return carry

        lax.fori_loop(0, n_chunks // 8, body, None)
        plsc.subcore_barrier()
        pltpu.sync_copy(acc.at[pl.ds(s * rpt, rpt)],
                        out.at[pl.ds(c * npad + s * rpt, rpt)])

    return k


def _tc_edge_mlp(epad, dh, de, be):
    grid = epad // be

    def body(hs, hd, ef, a, b, cc, b1, w2, b2, out):
        z = jnp.dot(hs[...], a[...], preferred_element_type=jnp.float32)
        z += jnp.dot(hd[...], b[...], preferred_element_type=jnp.float32)
        z += jnp.dot(ef[...], cc[...], preferred_element_type=jnp.float32)
        z = jnp.maximum(z + b1[...], 0.0)
        out[...] = jnp.dot(z, w2[...],
                           preferred_element_type=jnp.float32) + b2[...]

    def full(shape):
        return pl.BlockSpec(shape, lambda i: (0, 0))

    def make(a, b, cc, b1, w2, b2):
        call = pl.pallas_call(
            body,
            grid=(grid,),
            in_specs=[
                pl.BlockSpec((be, dh), lambda i: (i, 0)),
                pl.BlockSpec((be, dh), lambda i: (i, 0)),
                pl.BlockSpec((be, de), lambda i: (i, 0)),
                full(a.shape), full(b.shape), full(cc.shape),
                full(b1.shape), full(w2.shape), full(b2.shape),
            ],
            out_specs=pl.BlockSpec((be, 32), lambda i: (i, 0)),
            out_shape=jax.ShapeDtypeStruct((epad, 32), jnp.float32),
        )
        return lambda hs, hd, ef: call(hs, hd, ef, a, b, cc, b1, w2, b2)

    return make


def _tc_node_mlp(npad, dh, bn):
    grid = npad // bn

    def body(h, aggp, d1, d2, b1, w2, b2, out):
        agg = aggp[0] + aggp[1]
        z = jnp.dot(h[...], d1[...], preferred_element_type=jnp.float32)
        z += jnp.dot(agg, d2[...], preferred_element_type=jnp.float32)
        z = jnp.maximum(z + b1[...], 0.0)
        out[...] = jnp.dot(z, w2[...],
                           preferred_element_type=jnp.float32) + b2[...]

    def full(shape):
        return pl.BlockSpec(shape, lambda i: tuple(0 for _ in shape))

    def make(d1, d2, b1, w2, b2):
        call = pl.pallas_call(
            body,
            grid=(grid,),
            in_specs=[
                pl.BlockSpec((bn, dh), lambda i: (i, 0)),
                pl.BlockSpec((NC, bn, 32), lambda i: (0, i, 0)),
                full(d1.shape), full(d2.shape), full(b1.shape),
                full(w2.shape), full(b2.shape),
            ],
            out_specs=pl.BlockSpec((bn, 32), lambda i: (i, 0)),
            out_shape=jax.ShapeDtypeStruct((npad, 32), jnp.float32),
        )
        return lambda h, aggp: call(h, aggp, d1, d2, b1, w2, b2)

    return make


def _tc_assemble(npad, bn):
    grid = npad // bn

    def body(sp, cp, h0, out):
        cnt = jnp.maximum((cp[0] + cp[1])[:, 0:1], 1.0)
        hm = (sp[0] + sp[1]) / cnt
        out[...] = jnp.concatenate(
            [hm, h0[:, 0:2], h0[:, 3:6], jnp.zeros((bn, 11), jnp.float32)],
            axis=1)

    return pl.pallas_call(
        body,
        grid=(grid,),
        in_specs=[
            pl.BlockSpec((NC, bn, 32), lambda i: (0, i, 0)),
            pl.BlockSpec((NC, bn, 16), lambda i: (0, i, 0)),
            pl.BlockSpec((bn, 16), lambda i: (i, 0)),
        ],
        out_specs=pl.BlockSpec((bn, 48), lambda i: (i, 0)),
        out_shape=jax.ShapeDtypeStruct((npad, 48), jnp.float32),
    )


def _tc_final(npad, bn):
    grid = npad // bn

    def body(sp, cp, h0, wa, wb, bd, out):
        cnt = jnp.maximum((cp[0] + cp[1])[:, 0:1], 1.0)
        hm = (sp[0] + sp[1]) / cnt
        z = jnp.dot(hm, wa[...], preferred_element_type=jnp.float32)
        z += jnp.dot(h0[:, 0:2], wb[...], preferred_element_type=jnp.float32)
        out[...] = z + bd[...]

    def full(shape):
        return pl.BlockSpec(shape, lambda i: (0, 0))

    def make(wa, wb, bd):
        call = pl.pallas_call(
            body,
            grid=(grid,),
            in_specs=[
                pl.BlockSpec((NC, bn, 32), lambda i: (0, i, 0)),
                pl.BlockSpec((NC, bn, 16), lambda i: (0, i, 0)),
                pl.BlockSpec((bn, 16), lambda i: (i, 0)),
                full(wa.shape), full(wb.shape), full(bd.shape),
            ],
            out_specs=pl.BlockSpec((bn, 8), lambda i: (i, 0)),
            out_shape=jax.ShapeDtypeStruct((npad, 8), jnp.float32),
        )
        return lambda sp, cp, h0: call(sp, cp, h0, wa, wb, bd)

    return make


def _pad_rows(w, rows):
    return jnp.concatenate(
        [w, jnp.zeros((rows - w.shape[0], w.shape[1]), w.dtype)], axis=0)


def kernel(x, edge_index, edge_attr, params):
    n = x.shape[0]
    e = edge_index.shape[1]
    npad = ((n + 16) + 127) // 128 * 128
    epad = -(-e // (NW * CHUNK * 8)) * (NW * CHUNK * 8)
    n_chunks_tot = epad // CHUNK
    bn = npad // 8
    be = 4096

    src = edge_index[0].astype(jnp.int32)
    dst = edge_index[1].astype(jnp.int32)
    src2 = jnp.concatenate(
        [src, jnp.zeros((epad - e,), jnp.int32)]).reshape(
        n_chunks_tot, CHUNK)
    dst2 = jnp.concatenate(
        [dst, jnp.full((epad - e,), n, jnp.int32)]).reshape(
        n_chunks_tot, CHUNK)
    ea = _pad_rows(edge_attr.astype(jnp.float32), epad)

    z32 = jnp.zeros((npad, 32), jnp.float32)
    z16 = jnp.zeros((npad, 16), jnp.float32)
    ones16 = jnp.ones((CHUNK, 16), jnp.float32)

    h0p = _pad_rows(jnp.concatenate(
        [x.astype(jnp.float32), jnp.zeros((n, 10), jnp.float32)], axis=1),
        npad)

    p0, p1 = params["proc0"], params["proc1"]
    row = lambda v: v.reshape(1, -1).astype(jnp.float32)
    f32 = lambda v: v.astype(jnp.float32)

    a0 = _pad_rows(f32(p0["We1"][0:6]), 16)
    b0 = _pad_rows(f32(p0["We1"][6:12]), 16)
    c0 = f32(p0["We1"][12:16])
    d1_0 = _pad_rows(f32(p0["Wn1"][0:6]), 16)
    d2_0 = f32(p0["Wn1"][6:38])
    a1 = _pad_rows(f32(p1["We1"][0:37]), 48)
    b1w = _pad_rows(f32(p1["We1"][37:74]), 48)
    c1 = f32(p1["We1"][74:106])
    d1_1 = _pad_rows(f32(p1["Wn1"][0:37]), 48)
    d2_1 = f32(p1["Wn1"][37:69])
    wda = jnp.concatenate(
        [f32(params["Wd"][0:32]), jnp.zeros((32, 5), jnp.float32)], axis=1)
    wdb = jnp.concatenate(
        [f32(params["Wd"][32:34]), jnp.zeros((2, 5), jnp.float32)], axis=1)
    bdp = jnp.concatenate(
        [row(params["bd"]), jnp.zeros((1, 5), jnp.float32)], axis=1)

    gather16 = _sc_gather2(npad, 16, epad)
    gather48 = _sc_gather2(npad, 48, epad)
    scat_vals = _sc_scatter_add(npad, 32, epad, gather_table=False)
    scat_gath = _sc_scatter_add(npad, 32, epad, gather_table=True)
    count_k = _sc_count(npad, epad)
    edge0 = _tc_edge_mlp(epad, 16, 4, be)(
        a0, b0, c0, row(p0["be1"]), f32(p0["We2"]), row(p0["be2"]))
    edge1 = _tc_edge_mlp(epad, 48, 32, be)(
        a1, b1w, c1, row(p1["be1"]), f32(p1["We2"]), row(p1["be2"]))
    node0 = _tc_node_mlp(npad, 16, bn)(
        d1_0, d2_0, row(p0["bn1"]), f32(p0["Wn2"]), row(p0["bn2"]))
    node1 = _tc_node_mlp(npad, 48, bn)(
        d1_1, d2_1, row(p1["bn1"]), f32(p1["Wn2"]), row(p1["bn2"]))
    assemble = _tc_assemble(npad, bn)
    final = _tc_final(npad, bn)(wda, wdb, bdp)

    as3 = lambda v, w: v.reshape(NC, npad, w)

    hs0, hd0 = gather16(h0p, src2, dst2)
    e1 = edge0(hs0, hd0, ea)
    agg0 = scat_vals(e1, dst2, z32, src2)
    h1 = node0(h0p, as3(agg0, 32))
    cnt = count_k(dst2, z16, ones16)
    s0 = scat_gath(h1, dst2, z32, src2)
    h1cat = assemble(as3(s0, 32), as3(cnt, 16), h0p)
    hs1, hd1 = gather48(h1cat, src2, dst2)
    e2 = edge1(hs1, hd1, e1)
    agg1 = scat_vals(e2, dst2, z32, src2)
    h2 = node1(h1cat, as3(agg1, 32))
    s1 = scat_gath(h2, dst2, z32, src2)
    out = final(as3(s1, 32), as3(cnt, 16), h0p)
    return out[:n, :3]


# speedup vs baseline: 3.4642x; 1.1586x over previous
import functools

import jax
import jax.numpy as jnp
from jax import lax
from jax.experimental import pallas as pl
from jax.experimental.pallas import tpu as pltpu
from jax.experimental.pallas import tpu_sc as plsc

NC = 2
NS = 16
NW = NC * NS
CHUNK = 128
GK = 4


def _mesh():
    return plsc.VectorSubcoreMesh(core_axis_name="c", subcore_axis_name="s")


_SC_PARAMS = pltpu.CompilerParams(use_tc_tiling_on_sc=False)


def _sc_gather2(npad, d, epad):
    per_w = epad // NW
    n_chunks = per_w // CHUNK
    gsz = 4 * CHUNK
    rr = 40 * CHUNK
    nb = per_w // rr
    ops = rr // gsz

    @functools.partial(
        pl.kernel,
        out_type=(jax.ShapeDtypeStruct((epad, d), jnp.float32),
                  jax.ShapeDtypeStruct((epad, d), jnp.float32)),
        mesh=_mesh(), compiler_params=_SC_PARAMS,
        scratch_types=[
            pltpu.VMEM((rr,), jnp.int32),
            pltpu.VMEM((rr,), jnp.int32),
            pltpu.VMEM((2, gsz, d), jnp.float32),
            pltpu.VMEM((2, gsz, d), jnp.float32),
            [pltpu.SemaphoreType.DMA] * 2,
            [pltpu.SemaphoreType.DMA] * 2,
        ],
    )
    def k(table, src1, dst1, hs, hd, src_v, dst_v, rs_v, rd_v, gsem, wsem):
        c = lax.axis_index("c")
        s = lax.axis_index("s")
        w = s * NC + c
        base = w * per_w

        def fire_gather(o, slot):
            pltpu.async_copy(table.at[src_v.at[pl.ds(o * gsz, gsz)]],
                             rs_v.at[slot], gsem[slot])
            pltpu.async_copy(table.at[dst_v.at[pl.ds(o * gsz, gsz)]],
                             rd_v.at[slot], gsem[slot])

        def drain(ref, buf, sem):
            pltpu.make_async_copy(ref.at[pl.ds(0, gsz)], buf, sem).wait()

        def outer(ob, carry):
            pltpu.sync_copy(dst1.at[pl.ds(base + ob * rr, rr)], dst_v)
            pltpu.sync_copy(src1.at[pl.ds(base + ob * rr, rr)], src_v)
            fire_gather(0, 0)

            def body(ip, carry2):
                for b in range(2):
                    o = ip * 2 + b
                    row = base + ob * rr + o * gsz
                    drain(hs, rs_v.at[b], gsem[b])
                    drain(hd, rd_v.at[b], gsem[b])
                    pltpu.async_copy(rs_v.at[b], hs.at[pl.ds(row, gsz)],
                                     wsem[b])
                    pltpu.async_copy(rd_v.at[b], hd.at[pl.ds(row, gsz)],
                                     wsem[b])
                    b1 = (b + 1) % 2

                    @pl.when(o + 1 < ops)
                    def _():
                        @pl.when(o >= 1)
                        def _():
                            drain(hs, rs_v.at[b1], wsem[b1])
                            drain(hd, rd_v.at[b1], wsem[b1])
                        fire_gather(o + 1, b1)
                return carry2

            lax.fori_loop(0, ops // 2, body, None)
            for b in range(2):
                drain(hs, rs_v.at[b], wsem[b])
                drain(hd, rd_v.at[b], wsem[b])
            return carry

        lax.fori_loop(0, nb, outer, None)

    return k


def _sc_scatter_add(npad, width, epad, gather_table=False):
    per_w = epad // NW
    n_chunks = per_w // CHUNK
    rpt = npad // NS
    rr = 40
    nb = n_chunks // rr

    out_t = jax.ShapeDtypeStruct((NC * npad, width), jnp.float32)
    scratch = [
        pltpu.VMEM((rr, CHUNK), jnp.int32),
        pltpu.VMEM((rr, CHUNK), jnp.int32),
        pltpu.VMEM((4, CHUNK, width), jnp.float32),
        pltpu.VMEM_SHARED((npad, width), jnp.float32),
        [pltpu.SemaphoreType.DMA] * 4,
        [pltpu.SemaphoreType.DMA] * 4,
    ]

    @functools.partial(pl.kernel, out_type=out_t, mesh=_mesh(),
                       compiler_params=_SC_PARAMS, scratch_types=scratch)
    def k(src_data, dst2, zeros, src2, out, dst_v, src_v, buf, acc,
          lsem, ssem):
        c = lax.axis_index("c")
        s = lax.axis_index("s")
        w = s * NC + c
        pltpu.sync_copy(zeros.at[pl.ds(s * rpt, rpt)],
                        acc.at[pl.ds(s * rpt, rpt)])
        plsc.subcore_barrier()
        base = w * per_w

        def start(jl, jg, b):
            if gather_table:
                pltpu.async_copy(src_data.at[src_v.at[jl]], buf.at[b],
                                 lsem[b])
            else:
                pltpu.async_copy(
                    src_data.at[pl.ds(base + jg * CHUNK, CHUNK)],
                    buf.at[b], lsem[b])

        def outer(ob, carry):
            pltpu.sync_copy(
                dst2.at[pl.ds(w * n_chunks + ob * rr, rr)], dst_v)
            if gather_table:
                pltpu.sync_copy(
                    src2.at[pl.ds(w * n_chunks + ob * rr, rr)], src_v)
            start(0, ob * rr, 0)
            start(1, ob * rr + 1, 1)

            def body(g, carry2):
                for b in range(4):
                    jl = g * 4 + b
                    b2 = (b + 2) % 4
                    pltpu.make_async_copy(src_data.at[pl.ds(0, CHUNK)],
                                          buf.at[b], lsem[b]).wait()
                    pltpu.async_copy(buf.at[b], acc.at[dst_v.at[jl]],
                                     ssem[b], add=True)

                    @pl.when(jl + 2 < rr)
                    def _():
                        @pl.when(jl >= 2)
                        def _():
                            pltpu.make_async_copy(
                                src_data.at[pl.ds(0, CHUNK)],
                                buf.at[b2], ssem[b2]).wait()
                        start(jl + 2, ob * rr + jl + 2, b2)
                return carry2

            lax.fori_loop(0, rr // 4, body, None)
            for b in range(4):
                pltpu.make_async_copy(src_data.at[pl.ds(0, CHUNK)],
                                      buf.at[b], ssem[b]).wait()
            return carry

        lax.fori_loop(0, nb, outer, None)
        plsc.subcore_barrier()
        pltpu.sync_copy(acc.at[pl.ds(s * rpt, rpt)],
                        out.at[pl.ds(c * npad + s * rpt, rpt)])

    return k


def _sc_count(npad, epad):
    per_w = epad // NW
    n_chunks = per_w // CHUNK
    rpt = npad // NS
    width = 16

    @functools.partial(
        pl.kernel,
        out_type=jax.ShapeDtypeStruct((NC * npad, width), jnp.float32),
        mesh=_mesh(), compiler_params=_SC_PARAMS,
        scratch_types=[
            pltpu.VMEM((n_chunks * CHUNK,), jnp.int32),
            pltpu.VMEM((8 * CHUNK, width), jnp.float32),
            pltpu.VMEM_SHARED((npad, width), jnp.float32),
            pltpu.SemaphoreType.DMA,
        ],
    )
    def k(dst1, zeros, ones, out, dst_v, ones_v, acc, sem):
        c = lax.axis_index("c")
        s = lax.axis_index("s")
        w = s * NC + c
        gsz = 8 * CHUNK
        pltpu.sync_copy(dst1.at[pl.ds(w * per_w, per_w)], dst_v)
        pltpu.sync_copy(ones, ones_v)
        pltpu.sync_copy(zeros.at[pl.ds(s * rpt, rpt)],
                        acc.at[pl.ds(s * rpt, rpt)])
        plsc.subcore_barrier()

        ng = per_w // gsz

        def body(g, carry):
            pltpu.async_copy(ones_v, acc.at[dst_v.at[pl.ds(g * gsz, gsz)]],
                             sem, add=True)

            @pl.when(g >= 2)
            def _():
                pltpu.make_async_copy(ones, ones_v, sem).wait()
            return carry

        lax.fori_loop(0, ng, body, None)
        pltpu.make_async_copy(ones, ones_v, sem).wait()
        pltpu.make_async_copy(ones, ones_v, sem).wait()
        plsc.subcore_barrier()
        pltpu.sync_copy(acc.at[pl.ds(s * rpt, rpt)],
                        out.at[pl.ds(c * npad + s * rpt, rpt)])

    return k


def _tc_edge_mlp(epad, dh, de, be, ef_t=False):
    grid = epad // be

    def body(hs, hd, ef, a, b, cc, b1, w2, b2, out):
        z = jnp.dot(hs[...], a[...], preferred_element_type=jnp.float32)
        z += jnp.dot(hd[...], b[...], preferred_element_type=jnp.float32)
        if ef_t:
            z += lax.dot_general(ef[...], cc[...], (((0,), (0,)), ((), ())),
                                 preferred_element_type=jnp.float32)
        else:
            z += jnp.dot(ef[...], cc[...],
                         preferred_element_type=jnp.float32)
        z = jnp.maximum(z + b1[...], 0.0)
        out[...] = jnp.dot(z, w2[...],
                           preferred_element_type=jnp.float32) + b2[...]

    def full(shape):
        return pl.BlockSpec(shape, lambda i: (0, 0))

    ef_spec = (pl.BlockSpec((de, be), lambda i: (0, i)) if ef_t
               else pl.BlockSpec((be, de), lambda i: (i, 0)))

    def make(a, b, cc, b1, w2, b2):
        call = pl.pallas_call(
            body,
            grid=(grid,),
            in_specs=[
                pl.BlockSpec((be, dh), lambda i: (i, 0)),
                pl.BlockSpec((be, dh), lambda i: (i, 0)),
                ef_spec,
                full(a.shape), full(b.shape), full(cc.shape),
                full(b1.shape), full(w2.shape), full(b2.shape),
            ],
            out_specs=pl.BlockSpec((be, 32), lambda i: (i, 0)),
            out_shape=jax.ShapeDtypeStruct((epad, 32), jnp.float32),
        )
        return lambda hs, hd, ef: call(hs, hd, ef, a, b, cc, b1, w2, b2)

    return make


def _tc_node_mlp(npad, dh, bn):
    grid = npad // bn

    def body(h, aggp, d1, d2, b1, w2, b2, out):
        agg = aggp[0] + aggp[1]
        z = jnp.dot(h[...], d1[...], preferred_element_type=jnp.float32)
        z += jnp.dot(agg, d2[...], preferred_element_type=jnp.float32)
        z = jnp.maximum(z + b1[...], 0.0)
        out[...] = jnp.dot(z, w2[...],
                           preferred_element_type=jnp.float32) + b2[...]

    def full(shape):
        return pl.BlockSpec(shape, lambda i: tuple(0 for _ in shape))

    def make(d1, d2, b1, w2, b2):
        call = pl.pallas_call(
            body,
            grid=(grid,),
            in_specs=[
                pl.BlockSpec((bn, dh), lambda i: (i, 0)),
                pl.BlockSpec((NC, bn, 32), lambda i: (0, i, 0)),
                full(d1.shape), full(d2.shape), full(b1.shape),
                full(w2.shape), full(b2.shape),
            ],
            out_specs=pl.BlockSpec((bn, 32), lambda i: (i, 0)),
            out_shape=jax.ShapeDtypeStruct((npad, 32), jnp.float32),
        )
        return lambda h, aggp: call(h, aggp, d1, d2, b1, w2, b2)

    return make


def _tc_assemble(npad, bn):
    grid = npad // bn

    def body(sp, cp, h0, out):
        cnt = jnp.maximum((cp[0] + cp[1])[:, 0:1], 1.0)
        hm = (sp[0] + sp[1]) / cnt
        out[...] = jnp.concatenate(
            [hm, h0[:, 0:2], h0[:, 3:6], jnp.zeros((bn, 11), jnp.float32)],
            axis=1)

    return pl.pallas_call(
        body,
        grid=(grid,),
        in_specs=[
            pl.BlockSpec((NC, bn, 32), lambda i: (0, i, 0)),
            pl.BlockSpec((NC, bn, 16), lambda i: (0, i, 0)),
            pl.BlockSpec((bn, 16), lambda i: (i, 0)),
        ],
        out_specs=pl.BlockSpec((bn, 48), lambda i: (i, 0)),
        out_shape=jax.ShapeDtypeStruct((npad, 48), jnp.float32),
    )


def _tc_final(npad, bn):
    grid = npad // bn

    def body(sp, cp, h0, wa, wb, bd, out):
        cnt = jnp.maximum((cp[0] + cp[1])[:, 0:1], 1.0)
        hm = (sp[0] + sp[1]) / cnt
        z = jnp.dot(hm, wa[...], preferred_element_type=jnp.float32)
        z += jnp.dot(h0[:, 0:2], wb[...], preferred_element_type=jnp.float32)
        out[...] = z + bd[...]

    def full(shape):
        return pl.BlockSpec(shape, lambda i: (0, 0))

    def make(wa, wb, bd):
        call = pl.pallas_call(
            body,
            grid=(grid,),
            in_specs=[
                pl.BlockSpec((NC, bn, 32), lambda i: (0, i, 0)),
                pl.BlockSpec((NC, bn, 16), lambda i: (0, i, 0)),
                pl.BlockSpec((bn, 16), lambda i: (i, 0)),
                full(wa.shape), full(wb.shape), full(bd.shape),
            ],
            out_specs=pl.BlockSpec((bn, 8), lambda i: (i, 0)),
            out_shape=jax.ShapeDtypeStruct((npad, 8), jnp.float32),
        )
        return lambda sp, cp, h0: call(sp, cp, h0, wa, wb, bd)

    return make


def _pad_rows(w, rows):
    return jnp.concatenate(
        [w, jnp.zeros((rows - w.shape[0], w.shape[1]), w.dtype)], axis=0)


def kernel(x, edge_index, edge_attr, params):
    n = x.shape[0]
    e = edge_index.shape[1]
    npad = ((n + 16) + 127) // 128 * 128
    epad = -(-e // (NW * CHUNK * 8)) * (NW * CHUNK * 8)
    n_chunks_tot = epad // CHUNK
    bn = npad // 8
    be = 4096

    src = edge_index[0].astype(jnp.int32)
    dst = edge_index[1].astype(jnp.int32)
    src1 = jnp.concatenate([src, jnp.zeros((epad - e,), jnp.int32)])
    dst1 = jnp.concatenate([dst, jnp.full((epad - e,), n, jnp.int32)])
    src2 = src1.reshape(n_chunks_tot, CHUNK)
    dst2 = dst1.reshape(n_chunks_tot, CHUNK)
    eat = jnp.concatenate(
        [edge_attr.T.astype(jnp.float32),
         jnp.zeros((4, epad - e), jnp.float32)], axis=1)

    z32 = jnp.zeros((npad, 32), jnp.float32)
    z16 = jnp.zeros((npad, 16), jnp.float32)
    ones16 = jnp.ones((8 * CHUNK, 16), jnp.float32)

    h0p = _pad_rows(jnp.concatenate(
        [x.astype(jnp.float32), jnp.zeros((n, 10), jnp.float32)], axis=1),
        npad)

    p0, p1 = params["proc0"], params["proc1"]
    row = lambda v: v.reshape(1, -1).astype(jnp.float32)
    f32 = lambda v: v.astype(jnp.float32)

    a0 = _pad_rows(f32(p0["We1"][0:6]), 16)
    b0 = _pad_rows(f32(p0["We1"][6:12]), 16)
    c0 = f32(p0["We1"][12:16])
    d1_0 = _pad_rows(f32(p0["Wn1"][0:6]), 16)
    d2_0 = f32(p0["Wn1"][6:38])
    a1 = _pad_rows(f32(p1["We1"][0:37]), 48)
    b1w = _pad_rows(f32(p1["We1"][37:74]), 48)
    c1 = f32(p1["We1"][74:106])
    d1_1 = _pad_rows(f32(p1["Wn1"][0:37]), 48)
    d2_1 = f32(p1["Wn1"][37:69])
    wda = jnp.concatenate(
        [f32(params["Wd"][0:32]), jnp.zeros((32, 5), jnp.float32)], axis=1)
    wdb = jnp.concatenate(
        [f32(params["Wd"][32:34]), jnp.zeros((2, 5), jnp.float32)], axis=1)
    bdp = jnp.concatenate(
        [row(params["bd"]), jnp.zeros((1, 5), jnp.float32)], axis=1)

    gather16 = _sc_gather2(npad, 16, epad)
    gather48 = _sc_gather2(npad, 48, epad)
    scat_vals = _sc_scatter_add(npad, 32, epad, gather_table=False)
    scat_gath = _sc_scatter_add(npad, 32, epad, gather_table=True)
    count_k = _sc_count(npad, epad)
    edge0 = _tc_edge_mlp(epad, 16, 4, be, ef_t=True)(
        a0, b0, c0, row(p0["be1"]), f32(p0["We2"]), row(p0["be2"]))
    edge1 = _tc_edge_mlp(epad, 48, 32, be)(
        a1, b1w, c1, row(p1["be1"]), f32(p1["We2"]), row(p1["be2"]))
    node0 = _tc_node_mlp(npad, 16, bn)(
        d1_0, d2_0, row(p0["bn1"]), f32(p0["Wn2"]), row(p0["bn2"]))
    node1 = _tc_node_mlp(npad, 48, bn)(
        d1_1, d2_1, row(p1["bn1"]), f32(p1["Wn2"]), row(p1["bn2"]))
    assemble = _tc_assemble(npad, bn)
    final = _tc_final(npad, bn)(wda, wdb, bdp)

    as3 = lambda v, w: v.reshape(NC, npad, w)

    hs0, hd0 = gather16(h0p, src1, dst1)
    e1 = edge0(hs0, hd0, eat)
    agg0 = scat_vals(e1, dst2, z32, src2)
    h1 = node0(h0p, as3(agg0, 32))
    cnt = count_k(dst1, z16, ones16)
    s0 = scat_gath(h1, dst2, z32, src2)
    h1cat = assemble(as3(s0, 32), as3(cnt, 16), h0p)
    hs1, hd1 = gather48(h1cat, src1, dst1)
    e2 = edge1(hs1, hd1, e1)
    agg1 = scat_vals(e2, dst2, z32, src2)
    h2 = node1(h1cat, as3(agg1, 32))
    s1 = scat_gath(h2, dst2, z32, src2)
    out = final(as3(s1, 32), as3(cnt, 16), h0p)
    return out[:n, :3]
